# Initial kernel scaffold; baseline (speedup 1.0000x reference)
import jax
import jax.numpy as jnp
from jax.experimental import pallas as pl


def kernel(x, Wg, W1, W3, W2):
    raise NotImplementedError("write your pallas kernel here")


# speedup vs baseline: 5.1152x; 5.1152x over previous
import functools

import jax
import jax.numpy as jnp
from jax.experimental import pallas as pl
from jax.experimental.pallas import tpu as pltpu
from jax.experimental.pallas import tpu_sc as plsc

B, S, D = 1, 2048, 768
E, K = 16, 2
H = int(D * (5.0 / 3.0))
T = B * S
Z_LOSS_W = 0.001

BT = 128
NB = 48
PAD = NB * BT
NC, NS = 2, 16
NW = NC * NS
TPW = T // NW


def _route_body(x_ref, wg_ref, d0_ref, d1_ref, w0r_ref, w1r_ref, be_ref,
                aux_ref, pm_ref):
    xf = x_ref[...]
    wg = wg_ref[...]
    logits = jnp.dot(xf, wg, preferred_element_type=jnp.float32)
    z_loss = jnp.mean(logits * logits)
    mx = jnp.max(logits, axis=1, keepdims=True)
    p = jnp.exp(logits - mx)
    probs = p / jnp.sum(p, axis=1, keepdims=True)
    pm = jnp.sum(probs, axis=0, keepdims=True) * (1.0 / T)
    aux = E * jnp.sum(pm * pm) + Z_LOSS_W * z_loss
    aux_ref[...] = jnp.broadcast_to(aux, (1, 1))
    pm_ref[...] = pm

    iota_e = jax.lax.broadcasted_iota(jnp.int32, (T, E), 1)
    i1 = jnp.min(jnp.where(logits == mx, iota_e, E), axis=1, keepdims=True)
    one0 = (iota_e == i1).astype(jnp.float32)
    l2 = jnp.where(iota_e == i1, -1e30, logits)
    m2 = jnp.max(l2, axis=1, keepdims=True)
    i2 = jnp.min(jnp.where(l2 == m2, iota_e, E), axis=1, keepdims=True)
    one1 = (iota_e == i2).astype(jnp.float32)
    a = jnp.exp(m2 - mx)
    w0 = 1.0 / (1.0 + a)
    w1 = a * w0
    w0r_ref[...] = jnp.broadcast_to(w0, (T, E))
    w1r_ref[...] = jnp.broadcast_to(w1, (T, E))

    sel = one0 + one1
    csum = sel
    k = 1
    while k < T:
        csum = csum + jnp.concatenate(
            [jnp.zeros((k, E), jnp.float32), csum[: T - k]], axis=0)
        k *= 2
    cexc = csum - sel
    counts = jnp.sum(sel, axis=0, keepdims=True)
    ncb = jnp.floor((counts + (BT - 1)) * (1.0 / BT))
    lt = (jax.lax.broadcasted_iota(jnp.int32, (E, E), 0)
          < jax.lax.broadcasted_iota(jnp.int32, (E, E), 1)).astype(jnp.float32)
    po = BT * jnp.dot(ncb, lt, preferred_element_type=jnp.float32)
    rank0 = jnp.sum(cexc * one0, axis=1, keepdims=True)
    rank1 = jnp.sum(cexc * one1, axis=1, keepdims=True)
    po0 = jnp.sum(po * one0, axis=1, keepdims=True)
    po1 = jnp.sum(po * one1, axis=1, keepdims=True)
    d0_ref[...] = (po0 + rank0).astype(jnp.int32)
    d1_ref[...] = (po1 + rank1).astype(jnp.int32)

    barr = jax.lax.broadcasted_iota(jnp.int32, (NB, E), 0).astype(
        jnp.float32) * float(BT)
    be = jnp.sum((jnp.broadcast_to(po, (NB, E)) <= barr).astype(jnp.float32),
                 axis=1, keepdims=True) - 1.0
    be_ref[...] = be.astype(jnp.int32)


_router = pl.pallas_call(
    _route_body,
    out_shape=(
        jax.ShapeDtypeStruct((T, 1), jnp.int32),
        jax.ShapeDtypeStruct((T, 1), jnp.int32),
        jax.ShapeDtypeStruct((T, E), jnp.float32),
        jax.ShapeDtypeStruct((T, E), jnp.float32),
        jax.ShapeDtypeStruct((NB, 1), jnp.int32),
        jax.ShapeDtypeStruct((1, 1), jnp.float32),
        jax.ShapeDtypeStruct((1, E), jnp.float32),
    ),
)


def _ffn_body(be_ref, xg_ref, w1_ref, w3_ref, w2_ref, y_ref):
    del be_ref
    xb = xg_ref[...]
    h1 = jnp.dot(xb, w1_ref[0], preferred_element_type=jnp.float32)
    h3 = jnp.dot(xb, w3_ref[0], preferred_element_type=jnp.float32)
    act = h1 * (1.0 / (1.0 + jnp.exp(-h1))) * h3
    y_ref[...] = jnp.dot(act, w2_ref[0], preferred_element_type=jnp.float32)


_ffn = pl.pallas_call(
    _ffn_body,
    grid_spec=pltpu.PrefetchScalarGridSpec(
        num_scalar_prefetch=1,
        grid=(NB,),
        in_specs=[
            pl.BlockSpec((BT, D), lambda b, be: (b, 0)),
            pl.BlockSpec((1, D, H), lambda b, be: (be[b], 0, 0)),
            pl.BlockSpec((1, D, H), lambda b, be: (be[b], 0, 0)),
            pl.BlockSpec((1, H, D), lambda b, be: (be[b], 0, 0)),
        ],
        out_specs=pl.BlockSpec((BT, D), lambda b, be: (b, 0)),
    ),
    out_shape=jax.ShapeDtypeStruct((PAD, D), jnp.float32),
)


def _dispatch_body(x_hbm, d0_hbm, d1_hbm, xg_hbm, xrows, d0v, d1v, sem0, sem1):
    wid = jax.lax.axis_index("s") * NC + jax.lax.axis_index("c")
    t0 = wid * TPW
    pltpu.sync_copy(d0_hbm.at[pl.ds(t0, TPW)], d0v)
    pltpu.sync_copy(d1_hbm.at[pl.ds(t0, TPW)], d1v)
    pltpu.sync_copy(x_hbm.at[pl.ds(t0, TPW)], xrows)
    c0 = pltpu.async_copy(xrows, xg_hbm.at[d0v], sem0)
    c1 = pltpu.async_copy(xrows, xg_hbm.at[d1v], sem1)
    c0.wait()
    c1.wait()


def _combine_body(y_hbm, d0_hbm, d1_hbm, w0_hbm, w1_hbm, out_hbm,
                  y0v, y1v, d0v, d1v, w0v, w1v, sem0, sem1):
    wid = jax.lax.axis_index("s") * NC + jax.lax.axis_index("c")
    t0 = wid * TPW
    pltpu.sync_copy(d0_hbm.at[pl.ds(t0, TPW)], d0v)
    pltpu.sync_copy(d1_hbm.at[pl.ds(t0, TPW)], d1v)
    pltpu.sync_copy(w0_hbm.at[pl.ds(t0, TPW)], w0v)
    pltpu.sync_copy(w1_hbm.at[pl.ds(t0, TPW)], w1v)
    c0 = pltpu.async_copy(y_hbm.at[d0v], y0v, sem0)
    c1 = pltpu.async_copy(y_hbm.at[d1v], y1v, sem1)
    c0.wait()
    c1.wait()

    def row(j, carry):
        w0 = w0v[j, :]
        w1 = w1v[j, :]
        for c in range(D // 16):
            sl = pl.ds(c * 16, 16)
            y0v[j, sl] = y0v[j, sl] * w0 + y1v[j, sl] * w1
        return carry

    jax.lax.fori_loop(0, TPW, row, 0)
    pltpu.sync_copy(y0v, out_hbm.at[pl.ds(t0, TPW)])


@functools.lru_cache(maxsize=1)
def _sc_kernels():
    mesh = plsc.VectorSubcoreMesh(core_axis_name="c", subcore_axis_name="s",
                                  num_cores=NC, num_subcores=NS)
    dispatch = pl.kernel(
        _dispatch_body,
        out_type=jax.ShapeDtypeStruct((PAD, D), jnp.float32),
        mesh=mesh,
        scratch_types=[
            pltpu.VMEM((TPW, D), jnp.float32),
            pltpu.VMEM((TPW,), jnp.int32),
            pltpu.VMEM((TPW,), jnp.int32),
            pltpu.SemaphoreType.DMA,
            pltpu.SemaphoreType.DMA,
        ],
    )
    combine = pl.kernel(
        _combine_body,
        out_type=jax.ShapeDtypeStruct((T, D), jnp.float32),
        mesh=mesh,
        scratch_types=[
            pltpu.VMEM((TPW, D), jnp.float32),
            pltpu.VMEM((TPW, D), jnp.float32),
            pltpu.VMEM((TPW,), jnp.int32),
            pltpu.VMEM((TPW,), jnp.int32),
            pltpu.VMEM((TPW, E), jnp.float32),
            pltpu.VMEM((TPW, E), jnp.float32),
            pltpu.SemaphoreType.DMA,
            pltpu.SemaphoreType.DMA,
        ],
    )
    return dispatch, combine


def kernel(x, Wg, W1, W3, W2):
    dispatch_sc, combine_sc = _sc_kernels()
    xf = x.reshape(T, D)
    d0, d1, w0r, w1r, be, aux, pm = _router(xf, Wg)
    d0 = d0.reshape(T)
    d1 = d1.reshape(T)
    be = be.reshape(NB)
    xg = dispatch_sc(xf, d0, d1)
    y = _ffn(be, xg, W1, W3, W2)
    out = combine_sc(y, d0, d1, w0r, w1r)
    return out.reshape(B, S, D), aux.reshape(()), pm.reshape(E)


# speedup vs baseline: 5.7048x; 1.1153x over previous
import functools

import jax
import jax.numpy as jnp
from jax.experimental import pallas as pl
from jax.experimental.pallas import tpu as pltpu
from jax.experimental.pallas import tpu_sc as plsc

B, S, D = 1, 2048, 768
E, K = 16, 2
H = int(D * (5.0 / 3.0))
T = B * S
Z_LOSS_W = 0.001

BT = 128
NB = 48
PAD = NB * BT
NC, NS = 2, 16
NW = NC * NS
TPW = T // NW


def _route_body(x_ref, wg_ref, d0_ref, d1_ref, w0r_ref, w1r_ref, rid_ref,
                rex_ref, nrun_ref, aux_ref, pm_ref):
    xf = x_ref[0]
    wg = wg_ref[...]
    logits = jnp.dot(xf, wg, preferred_element_type=jnp.float32)
    z_loss = jnp.mean(logits * logits)
    mx = jnp.max(logits, axis=1, keepdims=True)
    p = jnp.exp(logits - mx)
    probs = p / jnp.sum(p, axis=1, keepdims=True)
    pm = jnp.sum(probs, axis=0, keepdims=True) * (1.0 / T)
    aux = E * jnp.sum(pm * pm) + Z_LOSS_W * z_loss
    aux_ref[...] = jnp.broadcast_to(aux, (1, 1))
    pm_ref[...] = pm

    iota_e = jax.lax.broadcasted_iota(jnp.int32, (T, E), 1)
    i1 = jnp.min(jnp.where(logits == mx, iota_e, E), axis=1, keepdims=True)
    one0 = (iota_e == i1).astype(jnp.float32)
    l2 = jnp.where(iota_e == i1, -1e30, logits)
    m2 = jnp.max(l2, axis=1, keepdims=True)
    i2 = jnp.min(jnp.where(l2 == m2, iota_e, E), axis=1, keepdims=True)
    one1 = (iota_e == i2).astype(jnp.float32)
    a = jnp.exp(m2 - mx)
    w0 = 1.0 / (1.0 + a)
    w1 = a * w0
    w0r_ref[...] = jnp.broadcast_to(w0, (T, E))
    w1r_ref[...] = jnp.broadcast_to(w1, (T, E))

    sel = one0 + one1
    csum = sel
    k = 1
    while k < T:
        csum = csum + jnp.concatenate(
            [jnp.zeros((k, E), jnp.float32), csum[: T - k]], axis=0)
        k *= 2
    cexc = csum - sel
    counts = jnp.sum(sel, axis=0, keepdims=True)
    ncb = jnp.floor((counts + (BT - 1)) * (1.0 / BT))
    lt = (jax.lax.broadcasted_iota(jnp.int32, (E, E), 0)
          < jax.lax.broadcasted_iota(jnp.int32, (E, E), 1)).astype(jnp.float32)
    po = BT * jnp.dot(ncb, lt, preferred_element_type=jnp.float32)
    rank0 = jnp.sum(cexc * one0, axis=1, keepdims=True)
    rank1 = jnp.sum(cexc * one1, axis=1, keepdims=True)
    po0 = jnp.sum(po * one0, axis=1, keepdims=True)
    po1 = jnp.sum(po * one1, axis=1, keepdims=True)
    d0_ref[...] = (po0 + rank0).astype(jnp.int32).reshape(T)
    d1_ref[...] = (po1 + rank1).astype(jnp.int32).reshape(T)

    nonempty = (ncb > 0.0).astype(jnp.float32)
    barr = jax.lax.broadcasted_iota(jnp.int32, (NB, E), 0).astype(
        jnp.float32) * float(BT)
    hit = (jnp.broadcast_to(po, (NB, E)) <= barr).astype(jnp.float32)
    rid = jnp.sum(hit * jnp.broadcast_to(nonempty, (NB, E)),
                  axis=1, keepdims=True) - 1.0
    rid_ref[...] = rid.astype(jnp.int32).reshape(NB)
    erank = jnp.dot(nonempty, lt, preferred_element_type=jnp.float32)
    r_iota = jax.lax.broadcasted_iota(jnp.int32, (NB, E), 0).astype(jnp.float32)
    e_iota = jax.lax.broadcasted_iota(jnp.int32, (NB, E), 1).astype(jnp.float32)
    sel_r = (jnp.broadcast_to(erank, (NB, E)) == r_iota) * \
        jnp.broadcast_to(nonempty, (NB, E))
    rex_ref[...] = jnp.sum(sel_r * e_iota, axis=1,
                           keepdims=True).astype(jnp.int32).reshape(NB)
    nrun_ref[...] = jnp.broadcast_to(jnp.sum(nonempty), (1,)).astype(jnp.int32)


_router = pl.pallas_call(
    _route_body,
    out_shape=(
        jax.ShapeDtypeStruct((T,), jnp.int32),
        jax.ShapeDtypeStruct((T,), jnp.int32),
        jax.ShapeDtypeStruct((T, E), jnp.float32),
        jax.ShapeDtypeStruct((T, E), jnp.float32),
        jax.ShapeDtypeStruct((NB,), jnp.int32),
        jax.ShapeDtypeStruct((NB,), jnp.int32),
        jax.ShapeDtypeStruct((1,), jnp.int32),
        jax.ShapeDtypeStruct((1, 1), jnp.float32),
        jax.ShapeDtypeStruct((1, E), jnp.float32),
    ),
)


NSLOT = 3


def _ffn_body(rid_ref, rex_ref, nrun_ref, xg_ref, w1_hbm, w3_hbm, w2_hbm,
              y_ref, w1s, w3s, w2s, wsem):
    b = pl.program_id(0)
    nruns = nrun_ref[0]
    r = rid_ref[b]
    prev = jnp.where(b > 0, rid_ref[jnp.maximum(b - 1, 0)], -1)
    slot = jax.lax.rem(r, NSLOT)

    def start_run(rr):
        e = rex_ref[rr]
        sl = jax.lax.rem(rr, NSLOT)
        for lo, ln in ((0, D // 2), (D // 2, D // 2)):
            pltpu.make_async_copy(w1_hbm.at[e, pl.ds(lo, ln)],
                                  w1s.at[sl, pl.ds(lo, ln)],
                                  wsem.at[sl, 0]).start()
            pltpu.make_async_copy(w3_hbm.at[e, pl.ds(lo, ln)],
                                  w3s.at[sl, pl.ds(lo, ln)],
                                  wsem.at[sl, 1]).start()
        for lo, ln in ((0, H // 2), (H // 2, H // 2)):
            pltpu.make_async_copy(w2_hbm.at[e, pl.ds(lo, ln)],
                                  w2s.at[sl, pl.ds(lo, ln)],
                                  wsem.at[sl, 2]).start()

    @pl.when(b == 0)
    def _():
        start_run(0)

        @pl.when(nruns > 1)
        def _():
            start_run(1)

    @pl.when(r != prev)
    def _():
        @pl.when(r + 2 < nruns)
        def _():
            start_run(r + 2)

        e = rex_ref[r]
        pltpu.make_async_copy(w1_hbm.at[e], w1s.at[slot], wsem.at[slot, 0]).wait()
        pltpu.make_async_copy(w3_hbm.at[e], w3s.at[slot], wsem.at[slot, 1]).wait()
        pltpu.make_async_copy(w2_hbm.at[e], w2s.at[slot], wsem.at[slot, 2]).wait()

    xb = xg_ref[...]
    h1 = jnp.dot(xb, w1s[slot], preferred_element_type=jnp.float32)
    h3 = jnp.dot(xb, w3s[slot], preferred_element_type=jnp.float32)
    act = h1 * (1.0 / (1.0 + jnp.exp(-h1))) * h3
    y_ref[...] = jnp.dot(act, w2s[slot], preferred_element_type=jnp.float32)


_ffn = pl.pallas_call(
    _ffn_body,
    grid_spec=pltpu.PrefetchScalarGridSpec(
        num_scalar_prefetch=3,
        grid=(NB,),
        in_specs=[
            pl.BlockSpec((BT, D), lambda b, rid, rex, nr: (b, 0)),
            pl.BlockSpec(memory_space=pl.ANY),
            pl.BlockSpec(memory_space=pl.ANY),
            pl.BlockSpec(memory_space=pl.ANY),
        ],
        out_specs=pl.BlockSpec((BT, D), lambda b, rid, rex, nr: (b, 0)),
        scratch_shapes=[
            pltpu.VMEM((NSLOT, D, H), jnp.float32),
            pltpu.VMEM((NSLOT, D, H), jnp.float32),
            pltpu.VMEM((NSLOT, H, D), jnp.float32),
            pltpu.SemaphoreType.DMA((NSLOT, 3)),
        ],
    ),
    out_shape=jax.ShapeDtypeStruct((PAD, D), jnp.float32),
)


HC = TPW // 2


def _dispatch_body(x_hbm, d0_hbm, d1_hbm, xg_hbm, xrows, d0v, d1v,
                   sem0, sem1, sem2, sem3):
    wid = jax.lax.axis_index("s") * NC + jax.lax.axis_index("c")
    t0 = wid * TPW
    pltpu.sync_copy(d0_hbm.at[pl.ds(t0, HC)], d0v.at[0])
    pltpu.sync_copy(d0_hbm.at[pl.ds(t0 + HC, HC)], d0v.at[1])
    pltpu.sync_copy(d1_hbm.at[pl.ds(t0, HC)], d1v.at[0])
    pltpu.sync_copy(d1_hbm.at[pl.ds(t0 + HC, HC)], d1v.at[1])
    la = pltpu.async_copy(x_hbm.at[0, pl.ds(t0, HC)],
                          xrows.at[pl.ds(0, HC)], sem0)
    lb = pltpu.async_copy(x_hbm.at[0, pl.ds(t0 + HC, HC)],
                          xrows.at[pl.ds(HC, HC)], sem1)
    la.wait()
    sa0 = pltpu.async_copy(xrows.at[pl.ds(0, HC)], xg_hbm.at[d0v.at[0]], sem2)
    sa1 = pltpu.async_copy(xrows.at[pl.ds(0, HC)], xg_hbm.at[d1v.at[0]], sem3)
    lb.wait()
    sb0 = pltpu.async_copy(xrows.at[pl.ds(HC, HC)], xg_hbm.at[d0v.at[1]], sem2)
    sb1 = pltpu.async_copy(xrows.at[pl.ds(HC, HC)], xg_hbm.at[d1v.at[1]], sem3)
    sa0.wait()
    sa1.wait()
    sb0.wait()
    sb1.wait()


def _combine_body(y_hbm, d0_hbm, d1_hbm, w0_hbm, w1_hbm, out_hbm,
                  y0v, y1v, d0v, d1v, w0v, w1v, sem0, sem1, sem2, sem3):
    wid = jax.lax.axis_index("s") * NC + jax.lax.axis_index("c")
    t0 = wid * TPW
    pltpu.sync_copy(d0_hbm.at[pl.ds(t0, TPW)], d0v)
    pltpu.sync_copy(d1_hbm.at[pl.ds(t0, TPW)], d1v)
    c0a = pltpu.async_copy(y_hbm.at[d0v.at[pl.ds(0, HC)]],
                           y0v.at[pl.ds(0, HC)], sem0)
    c1a = pltpu.async_copy(y_hbm.at[d1v.at[pl.ds(0, HC)]],
                           y1v.at[pl.ds(0, HC)], sem1)
    c0b = pltpu.async_copy(y_hbm.at[d0v.at[pl.ds(HC, HC)]],
                           y0v.at[pl.ds(HC, HC)], sem2)
    c1b = pltpu.async_copy(y_hbm.at[d1v.at[pl.ds(HC, HC)]],
                           y1v.at[pl.ds(HC, HC)], sem3)
    pltpu.sync_copy(w0_hbm.at[pl.ds(t0, TPW)], w0v)
    pltpu.sync_copy(w1_hbm.at[pl.ds(t0, TPW)], w1v)

    def row(j, carry):
        w0 = w0v[j, :]
        w1 = w1v[j, :]
        for c in range(D // 16):
            sl = pl.ds(c * 16, 16)
            y0v[j, sl] = y0v[j, sl] * w0 + y1v[j, sl] * w1
        return carry

    c0a.wait()
    c1a.wait()
    jax.lax.fori_loop(0, HC, row, 0)
    sa = pltpu.async_copy(y0v.at[pl.ds(0, HC)], out_hbm.at[pl.ds(t0, HC)],
                          sem0)
    c0b.wait()
    c1b.wait()
    jax.lax.fori_loop(HC, TPW, row, 0)
    pltpu.sync_copy(y0v.at[pl.ds(HC, HC)], out_hbm.at[pl.ds(t0 + HC, HC)])
    sa.wait()


@functools.lru_cache(maxsize=1)
def _sc_kernels():
    mesh = plsc.VectorSubcoreMesh(core_axis_name="c", subcore_axis_name="s",
                                  num_cores=NC, num_subcores=NS)
    dispatch = pl.kernel(
        _dispatch_body,
        out_type=jax.ShapeDtypeStruct((PAD, D), jnp.float32),
        mesh=mesh,
        scratch_types=[
            pltpu.VMEM((TPW, D), jnp.float32),
            pltpu.VMEM((2, HC), jnp.int32),
            pltpu.VMEM((2, HC), jnp.int32),
            pltpu.SemaphoreType.DMA,
            pltpu.SemaphoreType.DMA,
            pltpu.SemaphoreType.DMA,
            pltpu.SemaphoreType.DMA,
        ],
    )
    combine = pl.kernel(
        _combine_body,
        out_type=jax.ShapeDtypeStruct((T, D), jnp.float32),
        mesh=mesh,
        scratch_types=[
            pltpu.VMEM((TPW, D), jnp.float32),
            pltpu.VMEM((TPW, D), jnp.float32),
            pltpu.VMEM((TPW,), jnp.int32),
            pltpu.VMEM((TPW,), jnp.int32),
            pltpu.VMEM((TPW, E), jnp.float32),
            pltpu.VMEM((TPW, E), jnp.float32),
            pltpu.SemaphoreType.DMA,
            pltpu.SemaphoreType.DMA,
            pltpu.SemaphoreType.DMA,
            pltpu.SemaphoreType.DMA,
        ],
    )
    return dispatch, combine


def kernel(x, Wg, W1, W3, W2):
    dispatch_sc, combine_sc = _sc_kernels()
    d0, d1, w0r, w1r, rid, rex, nrun, aux, pm = _router(x, Wg)
    xg = dispatch_sc(x, d0, d1)
    y = _ffn(rid, rex, nrun, xg, W1, W3, W2)
    out = combine_sc(y, d0, d1, w0r, w1r)
    return out.reshape(B, S, D), aux.reshape(()), pm.reshape(E)
